# baseline (speedup 1.0000x reference)
import functools

import jax
import jax.numpy as jnp
from jax import lax
from jax.experimental import pallas as pl
from jax.experimental.pallas import tpu as pltpu
from jax.experimental.pallas import tpu_sc as plsc

B = 16384
D = 32
TROWS = 8
NTILES = 125000
NC = 2
NS = 16
L = 16
NW = NC * NS
BPW = B // NW
IDXW = 128
NIDX = BPW // IDXW
WAVE = 32
NWAVES = BPW // WAVE
GLOBAL_MEAN = 3.82

_mesh = plsc.VectorSubcoreMesh(core_axis_name="c", subcore_axis_name="s")


@functools.partial(
    pl.kernel,
    mesh=_mesh,
    compiler_params=pltpu.CompilerParams(
        needs_layout_passes=False, use_tc_tiling_on_sc=True),
    out_type=jax.ShapeDtypeStruct((B,), jnp.float32),
    scratch_types=[
        pltpu.VMEM((NIDX, IDXW), jnp.int32),
        pltpu.VMEM((NIDX, IDXW), jnp.int32),
        pltpu.VMEM((2, L, TROWS, D), jnp.float32),
        pltpu.VMEM((2, L, TROWS, D), jnp.float32),
        pltpu.VMEM((BPW,), jnp.float32),
        pltpu.SemaphoreType.DMA,
        pltpu.SemaphoreType.DMA,
    ],
)
def _factors_sc(sid_hbm, pid_hbm, sf_hbm, pf_hbm, out_hbm,
                sid_v, pid_v, st_v, pt_v, out_v, sem0, sem1):
    wid = lax.axis_index("s") * NC + lax.axis_index("c")
    base = wid * BPW
    sf3 = sf_hbm.reshape(NTILES, TROWS, D)
    pf3 = pf_hbm.reshape(NTILES, TROWS, D)

    for j in range(NIDX):
        blk = wid * NIDX + j
        pltpu.sync_copy(sid_hbm.at[blk // 8, blk % 8], sid_v.at[j])
        pltpu.sync_copy(pid_hbm.at[blk // 8, blk % 8], pid_v.at[j])

    lanes = lax.iota(jnp.int32, L)
    NG = BPW // L

    def load_ids(g):
        j = g // (IDXW // L)
        o = (g - j * (IDXW // L)) * L
        return sid_v[j, pl.ds(o, L)], pid_v[j, pl.ds(o, L)]

    def fire(g, buf, sem):
        ids_s, ids_p = load_ids(g)
        sq = ids_s // TROWS
        pq = ids_p // TROWS
        for e in range(L):
            pltpu.async_copy(sf3.at[sq[e]], st_v.at[buf, e], sem)
            pltpu.async_copy(pf3.at[pq[e]], pt_v.at[buf, e], sem)

    def drain(sem):
        for e in range(L):
            pltpu.make_async_copy(sf3.at[0], st_v.at[0, 0], sem).wait()
            pltpu.make_async_copy(pf3.at[0], pt_v.at[0, 0], sem).wait()

    def compute(g, buf):
        ids_s, ids_p = load_ids(g)
        srow = ids_s - (ids_s // TROWS) * TROWS
        prow = ids_p - (ids_p // TROWS) * TROWS
        bvec = jnp.full((L,), buf, jnp.int32)
        acc = jnp.zeros((L,), jnp.float32)
        for d in range(D):
            dv = jnp.full((L,), d, jnp.int32)
            sv = plsc.load_gather(st_v, [bvec, lanes, srow, dv])
            pv = plsc.load_gather(pt_v, [bvec, lanes, prow, dv])
            acc = acc + sv * pv
        out_v[pl.ds(g * L, L)] = acc

    fire(0, 0, sem0)

    def pair_body(h, carry):
        g0 = h * 2

        @pl.when(g0 + 1 < NG)
        def _():
            fire(g0 + 1, 1, sem1)

        drain(sem0)
        compute(g0, 0)

        @pl.when(g0 + 2 < NG)
        def _():
            fire(g0 + 2, 0, sem0)

        drain(sem1)
        compute(g0 + 1, 1)
        return carry

    lax.fori_loop(0, NG // 2, pair_body, 0)

    pltpu.sync_copy(out_v, out_hbm.at[pl.ds(base, BPW)])


@functools.partial(
    pl.kernel,
    mesh=_mesh,
    compiler_params=pltpu.CompilerParams(
        needs_layout_passes=False, use_tc_tiling_on_sc=False),
    out_type=jax.ShapeDtypeStruct((B,), jnp.float32),
    scratch_types=[
        pltpu.VMEM((NIDX, IDXW), jnp.int32),
        pltpu.VMEM((NIDX, IDXW), jnp.int32),
        pltpu.VMEM((BPW,), jnp.float32),
        pltpu.VMEM((BPW,), jnp.float32),
        pltpu.VMEM((BPW,), jnp.float32),
        pltpu.SemaphoreType.DMA,
    ],
)
def _bias_sc(part_hbm, sid_hbm, pid_hbm, sb_hbm, pb_hbm, out_hbm,
             sid_v, pid_v, sb_v, pb_v, acc_v, sem):
    wid = lax.axis_index("s") * NC + lax.axis_index("c")
    base = wid * BPW

    pltpu.sync_copy(sid_hbm.at[wid], sid_v)
    pltpu.sync_copy(pid_hbm.at[wid], pid_v)
    pltpu.sync_copy(part_hbm.at[pl.ds(base, BPW)], acc_v)

    copies = []
    for j in range(NIDX):
        rows = pl.ds(j * IDXW, IDXW)
        copies.append(pltpu.async_copy(sb_hbm.at[0].at[sid_v.at[j]],
                                       sb_v.at[rows], sem))
        copies.append(pltpu.async_copy(pb_hbm.at[0].at[pid_v.at[j]],
                                       pb_v.at[rows], sem))
    for c in copies:
        c.wait()

    def body(g, carry):
        sl = pl.ds(g * L, L)
        acc_v[sl] = acc_v[sl] + sb_v[sl] + pb_v[sl] + jnp.float32(GLOBAL_MEAN)
        return carry

    lax.fori_loop(0, BPW // L, body, 0)

    pltpu.sync_copy(acc_v, out_hbm.at[pl.ds(base, BPW)])


def kernel(scientist_ids, paper_ids, scientist_factors, paper_factors,
           scientist_bias, paper_bias):
    sid3 = scientist_ids.reshape(16, 8, 128)
    pid3 = paper_ids.reshape(16, 8, 128)
    part = _factors_sc(sid3, pid3, scientist_factors, paper_factors)
    sid4 = scientist_ids.reshape(NW, NIDX, IDXW)
    pid4 = paper_ids.reshape(NW, NIDX, IDXW)
    sb = scientist_bias.T
    pb = paper_bias.T
    return _bias_sc(part, sid4, pid4, sb, pb)


# speedup vs baseline: 1.6173x; 1.6173x over previous
import functools

import jax
import jax.numpy as jnp
from jax import lax
from jax.experimental import pallas as pl
from jax.experimental.pallas import tpu as pltpu
from jax.experimental.pallas import tpu_sc as plsc

B = 16384
D = 32
TROWS = 8
NTILES = 125000
NC = 2
NS = 16
L = 16
NW = NC * NS
BPW = B // NW
IDXW = 128
NIDX = BPW // IDXW
WAVE = 32
NWAVES = BPW // WAVE
GLOBAL_MEAN = 3.82

_mesh = plsc.VectorSubcoreMesh(core_axis_name="c", subcore_axis_name="s")


@functools.partial(
    pl.kernel,
    mesh=_mesh,
    compiler_params=pltpu.CompilerParams(
        needs_layout_passes=False, use_tc_tiling_on_sc=True),
    out_type=jax.ShapeDtypeStruct((B,), jnp.float32),
    scratch_types=[
        pltpu.VMEM((NIDX, IDXW), jnp.int32),
        pltpu.VMEM((NIDX, IDXW), jnp.int32),
        pltpu.VMEM((2, L, TROWS, D), jnp.float32),
        pltpu.VMEM((2, L, TROWS, D), jnp.float32),
        pltpu.VMEM((BPW,), jnp.float32),
        pltpu.SemaphoreType.DMA,
        pltpu.SemaphoreType.DMA,
    ],
)
def _factors_sc(sid_hbm, pid_hbm, sf_hbm, pf_hbm, out_hbm,
                sid_v, pid_v, st_v, pt_v, out_v, sem0, sem1):
    wid = lax.axis_index("s") * NC + lax.axis_index("c")
    base = wid * BPW
    sf3 = sf_hbm
    pf3 = pf_hbm

    for j in range(NIDX):
        blk = wid * NIDX + j
        pltpu.sync_copy(sid_hbm.at[blk // 8, blk % 8], sid_v.at[j])
        pltpu.sync_copy(pid_hbm.at[blk // 8, blk % 8], pid_v.at[j])

    lanes = lax.iota(jnp.int32, L)
    NG = BPW // L

    def load_ids(g):
        j = g // (IDXW // L)
        o = (g - j * (IDXW // L)) * L
        return sid_v[j, pl.ds(o, L)], pid_v[j, pl.ds(o, L)]

    def fire(g, buf, sem):
        ids_s, ids_p = load_ids(g)
        sq = ids_s // TROWS
        pq = ids_p // TROWS
        for e in range(L):
            pltpu.async_copy(sf3.at[sq[e]], st_v.at[buf, e], sem)
            pltpu.async_copy(pf3.at[pq[e]], pt_v.at[buf, e], sem)

    def drain(sem):
        for e in range(L):
            pltpu.make_async_copy(sf3.at[0], st_v.at[0, 0], sem).wait()
            pltpu.make_async_copy(pf3.at[0], pt_v.at[0, 0], sem).wait()

    def compute(g, buf):
        ids_s, ids_p = load_ids(g)
        srow = ids_s - (ids_s // TROWS) * TROWS
        prow = ids_p - (ids_p // TROWS) * TROWS
        bvec = jnp.full((L,), buf, jnp.int32)
        acc = jnp.zeros((L,), jnp.float32)
        for d in range(D):
            dv = jnp.full((L,), d, jnp.int32)
            sv = plsc.load_gather(st_v, [bvec, lanes, srow, dv])
            pv = plsc.load_gather(pt_v, [bvec, lanes, prow, dv])
            acc = acc + sv * pv
        out_v[pl.ds(g * L, L)] = acc

    fire(0, 0, sem0)

    def pair_body(h, carry):
        g0 = h * 2

        @pl.when(g0 + 1 < NG)
        def _():
            fire(g0 + 1, 1, sem1)

        drain(sem0)
        compute(g0, 0)

        @pl.when(g0 + 2 < NG)
        def _():
            fire(g0 + 2, 0, sem0)

        drain(sem1)
        compute(g0 + 1, 1)
        return carry

    lax.fori_loop(0, NG // 2, pair_body, 0)

    pltpu.sync_copy(out_v, out_hbm.at[pl.ds(base, BPW)])


@functools.partial(
    pl.kernel,
    mesh=_mesh,
    compiler_params=pltpu.CompilerParams(
        needs_layout_passes=False, use_tc_tiling_on_sc=False),
    out_type=jax.ShapeDtypeStruct((B,), jnp.float32),
    scratch_types=[
        pltpu.VMEM((NIDX, IDXW), jnp.int32),
        pltpu.VMEM((NIDX, IDXW), jnp.int32),
        pltpu.VMEM((BPW,), jnp.float32),
        pltpu.VMEM((BPW,), jnp.float32),
        pltpu.VMEM((BPW,), jnp.float32),
        pltpu.SemaphoreType.DMA,
    ],
)
def _bias_sc(part_hbm, sid_hbm, pid_hbm, sb_hbm, pb_hbm, out_hbm,
             sid_v, pid_v, sb_v, pb_v, acc_v, sem):
    wid = lax.axis_index("s") * NC + lax.axis_index("c")
    base = wid * BPW

    pltpu.sync_copy(sid_hbm.at[wid], sid_v)
    pltpu.sync_copy(pid_hbm.at[wid], pid_v)
    pltpu.sync_copy(part_hbm.at[pl.ds(base, BPW)], acc_v)

    copies = []
    for j in range(NIDX):
        rows = pl.ds(j * IDXW, IDXW)
        copies.append(pltpu.async_copy(sb_hbm.at[0].at[sid_v.at[j]],
                                       sb_v.at[rows], sem))
        copies.append(pltpu.async_copy(pb_hbm.at[0].at[pid_v.at[j]],
                                       pb_v.at[rows], sem))
    for c in copies:
        c.wait()

    def body(g, carry):
        sl = pl.ds(g * L, L)
        acc_v[sl] = acc_v[sl] + sb_v[sl] + pb_v[sl] + jnp.float32(GLOBAL_MEAN)
        return carry

    lax.fori_loop(0, BPW // L, body, 0)

    pltpu.sync_copy(acc_v, out_hbm.at[pl.ds(base, BPW)])


def kernel(scientist_ids, paper_ids, scientist_factors, paper_factors,
           scientist_bias, paper_bias):
    sid3 = scientist_ids.reshape(16, 8, 128)
    pid3 = paper_ids.reshape(16, 8, 128)
    sf3 = scientist_factors.reshape(NTILES, TROWS, D)
    pf3 = paper_factors.reshape(NTILES, TROWS, D)
    part = _factors_sc(sid3, pid3, sf3, pf3)
    sid4 = scientist_ids.reshape(NW, NIDX, IDXW)
    pid4 = paper_ids.reshape(NW, NIDX, IDXW)
    sb = scientist_bias.T
    pb = paper_bias.T
    return _bias_sc(part, sid4, pid4, sb, pb)
